# baseline (speedup 1.0000x reference)
import functools

import jax
import jax.numpy as jnp
from jax import lax
from jax.experimental import pallas as pl
from jax.experimental.pallas import tpu as pltpu
from jax.experimental.pallas import tpu_sc as plsc

B = 4096
D = 128
NG = 200
WL = 50
H = 100
C = 10

NC = 2
NS = 16
NW = NC * NS
B_PER_W = B // NW
LANES = 16
NBUF = 4
CH0 = 128
CH1 = NG - CH0
OUT_TILE = 4

_mesh = plsc.VectorSubcoreMesh(core_axis_name="c", subcore_axis_name="s")


@functools.partial(
    pl.kernel,
    out_type=jax.ShapeDtypeStruct((B, D), jnp.float32),
    mesh=_mesh,
    scratch_types=[
        pltpu.VMEM((B_PER_W * NG,), jnp.int32),
        pltpu.VMEM((NBUF, NG, D), jnp.float32),
        pltpu.VMEM((2, OUT_TILE, D), jnp.float32),
        [pltpu.SemaphoreType.DMA] * (2 * NBUF),
        [pltpu.SemaphoreType.DMA] * 2,
    ],
)
def _ngram_sum(idx_hbm, table_hbm, out_hbm, idx_v, rows_v, out_v, sems,
               out_sems):
    wid = lax.axis_index("s") * NC + lax.axis_index("c")
    base = pl.multiple_of(wid * B_PER_W, B_PER_W)
    pltpu.sync_copy(idx_hbm.at[pl.ds(base * NG, B_PER_W * NG)], idx_v)

    def fire(r, h, b):
        off, n = (0, CH0) if h == 0 else (CH0, CH1)
        pltpu.async_copy(
            table_hbm.at[idx_v.at[pl.ds(pl.multiple_of(r * NG + off, 8), n)]],
            rows_v.at[b].at[pl.ds(off, n)], sems[2 * b + h])

    def drain(h, b):
        off, n = (0, CH0) if h == 0 else (CH0, CH1)
        pltpu.make_async_copy(
            table_hbm.at[pl.ds(0, n)],
            rows_v.at[b].at[pl.ds(off, n)], sems[2 * b + h]).wait()

    def accum_chunk(b, off, n, accs):
        def body(j, accs):
            a = tuple(
                accs[d] + rows_v[b, off + 2 * j, pl.ds(d * LANES, LANES)]
                for d in range(D // LANES))
            return tuple(
                a[d] + rows_v[b, off + 2 * j + 1, pl.ds(d * LANES, LANES)]
                for d in range(D // LANES))

        return lax.fori_loop(0, n // 2, body, accs)

    zeros = tuple(jnp.zeros((LANES,), jnp.float32) for _ in range(D // LANES))

    for b in range(NBUF):
        fire(b, 0, b)
        fire(b, 1, b)

    def step(r, b, ob, k):
        drain(0, b)
        accs = accum_chunk(b, 0, CH0, zeros)

        @pl.when(r + NBUF < B_PER_W)
        def _():
            fire(r + NBUF, 0, b)

        drain(1, b)
        accs = accum_chunk(b, CH0, CH1, accs)

        @pl.when(r + NBUF < B_PER_W)
        def _():
            fire(r + NBUF, 1, b)

        for d in range(D // LANES):
            out_v[ob, k, pl.ds(d * LANES, LANES)] = accs[d]

    ROWS_PER_IT = 2 * OUT_TILE
    N_IT = B_PER_W // ROWS_PER_IT

    def flush_wait(ob):
        pltpu.make_async_copy(
            out_v.at[ob], out_hbm.at[pl.ds(0, OUT_TILE)],
            out_sems[ob]).wait()

    def ring_body(rr, _):
        r0 = ROWS_PER_IT * rr
        for g in range(2):
            @pl.when(rr > 0)
            def _():
                flush_wait(g)

            for k in range(OUT_TILE):
                r = r0 + g * OUT_TILE + k
                step(r, (g * OUT_TILE + k) % NBUF, g, k)
            pltpu.async_copy(
                out_v.at[g],
                out_hbm.at[pl.ds(base + r0 + g * OUT_TILE, OUT_TILE)],
                out_sems[g])
        return 0

    lax.fori_loop(0, N_IT, ring_body, 0)
    flush_wait(0)
    flush_wait(1)


BB = 256


def _embs_sum_body(embs_ref, o_ref):
    o_ref[...] = jnp.sum(embs_ref[...], axis=0)


_embs_sum = pl.pallas_call(
    _embs_sum_body,
    grid=(B // BB,),
    in_specs=[pl.BlockSpec((WL, BB, D), lambda i: (0, i, 0))],
    out_specs=pl.BlockSpec((BB, D), lambda i: (i, 0)),
    out_shape=jax.ShapeDtypeStruct((B, D), jnp.float32),
)


def _combine_body(es_ref, ng_ref, w1_ref, b1_ref, w2_ref, b2_ref, o_ref):
    x = (es_ref[...] + ng_ref[...]) * (1.0 / (WL + NG))
    h = lax.dot_general(x, w1_ref[...], (((1,), (1,)), ((), ())),
                        preferred_element_type=jnp.float32) + b1_ref[...]
    logits_t = lax.dot_general(w2_ref[...], h, (((1,), (1,)), ((), ())),
                               preferred_element_type=jnp.float32) + b2_ref[...]
    o_ref[...] = jax.nn.sigmoid(logits_t)


_combine = pl.pallas_call(
    _combine_body,
    in_specs=[
        pl.BlockSpec((B, D), lambda: (0, 0)),
        pl.BlockSpec((B, D), lambda: (0, 0)),
        pl.BlockSpec((H, D), lambda: (0, 0)),
        pl.BlockSpec((1, H), lambda: (0, 0)),
        pl.BlockSpec((C, H), lambda: (0, 0)),
        pl.BlockSpec((C, 1), lambda: (0, 0)),
    ],
    out_specs=pl.BlockSpec((C, B), lambda: (0, 0)),
    out_shape=jax.ShapeDtypeStruct((C, B), jnp.float32),
)


def kernel(embs, ngram_embs, table, W_i2h, b_i2h, W_h2o, b_h2o):
    ng_sum = _ngram_sum(ngram_embs.astype(jnp.int32).reshape(-1), table)
    es = _embs_sum(jnp.transpose(embs, (1, 0, 2)))
    out_t = _combine(es, ng_sum, W_i2h, b_i2h.reshape(1, H),
                     W_h2o, b_h2o.reshape(C, 1))
    return jnp.transpose(out_t)


# speedup vs baseline: 1.0464x; 1.0464x over previous
import functools

import jax
import jax.numpy as jnp
from jax import lax
from jax.experimental import pallas as pl
from jax.experimental.pallas import tpu as pltpu
from jax.experimental.pallas import tpu_sc as plsc

B = 4096
D = 128
NG = 200
WL = 50
H = 100
C = 10

NC = 2
NS = 16
NW = NC * NS
B_PER_W = B // NW
LANES = 16
NBUF = 3
CH0 = 128
CH1 = NG - CH0

_mesh = plsc.VectorSubcoreMesh(core_axis_name="c", subcore_axis_name="s")


@functools.partial(
    pl.kernel,
    out_type=jax.ShapeDtypeStruct((B, D), jnp.float32),
    mesh=_mesh,
    scratch_types=[
        pltpu.VMEM((B_PER_W, NG), jnp.int32),
        pltpu.VMEM((NBUF, NG, D), jnp.float32),
        pltpu.VMEM((B_PER_W, D), jnp.float32),
        [pltpu.SemaphoreType.DMA] * (2 * NBUF),
    ],
)
def _ngram_sum(idx_hbm, table_hbm, out_hbm, idx_v, rows_v, out_v, sems):
    wid = lax.axis_index("s") * NC + lax.axis_index("c")
    base = pl.multiple_of(wid * B_PER_W, B_PER_W)
    pltpu.sync_copy(idx_hbm.at[pl.ds(base, B_PER_W)], idx_v)

    def fire(r, h, b):
        off, n = (0, CH0) if h == 0 else (CH0, CH1)
        pltpu.async_copy(
            table_hbm.at[idx_v.at[r, pl.ds(off, n)]],
            rows_v.at[b].at[pl.ds(off, n)], sems[2 * b + h])

    def drain(h, b):
        off, n = (0, CH0) if h == 0 else (CH0, CH1)
        pltpu.make_async_copy(
            table_hbm.at[pl.ds(0, n)],
            rows_v.at[b].at[pl.ds(off, n)], sems[2 * b + h]).wait()

    def accum_chunk(b, off, n, accs):
        def body(j, accs):
            a = tuple(
                accs[d] + rows_v[b, off + 2 * j, pl.ds(d * LANES, LANES)]
                for d in range(D // LANES))
            return tuple(
                a[d] + rows_v[b, off + 2 * j + 1, pl.ds(d * LANES, LANES)]
                for d in range(D // LANES))

        return lax.fori_loop(0, n // 2, body, accs)

    zeros = tuple(jnp.zeros((LANES,), jnp.float32) for _ in range(D // LANES))

    for b in range(NBUF):
        fire(b, 0, b)
        fire(b, 1, b)

    def step(r, b, refire):
        drain(0, b)
        accs = accum_chunk(b, 0, CH0, zeros)
        if refire:
            @pl.when(r + NBUF < B_PER_W)
            def _():
                fire(r + NBUF, 0, b)

        drain(1, b)
        accs = accum_chunk(b, CH0, CH1, accs)
        if refire:
            @pl.when(r + NBUF < B_PER_W)
            def _():
                fire(r + NBUF, 1, b)

        for d in range(D // LANES):
            out_v[r, pl.ds(d * LANES, LANES)] = accs[d]

    ROWS_PER_IT = 2 * NBUF
    N_IT = B_PER_W // ROWS_PER_IT

    def ring_body(rr, _):
        r0 = ROWS_PER_IT * rr
        for k in range(ROWS_PER_IT):
            step(r0 + k, k % NBUF, True)
        return 0

    lax.fori_loop(0, N_IT, ring_body, 0)
    for r in range(N_IT * ROWS_PER_IT, B_PER_W):
        step(r, r % NBUF, False)
    pltpu.sync_copy(out_v, out_hbm.at[pl.ds(base, B_PER_W)])


BB = 256


def _embs_sum_body(embs_ref, o_ref):
    o_ref[...] = jnp.sum(embs_ref[...], axis=0)


_embs_sum = pl.pallas_call(
    _embs_sum_body,
    grid=(B // BB,),
    in_specs=[pl.BlockSpec((WL, BB, D), lambda i: (0, i, 0))],
    out_specs=pl.BlockSpec((BB, D), lambda i: (i, 0)),
    out_shape=jax.ShapeDtypeStruct((B, D), jnp.float32),
)


def _combine_body(es_ref, ng_ref, w1_ref, b1_ref, w2_ref, b2_ref, o_ref):
    x = (es_ref[...] + ng_ref[...]) * (1.0 / (WL + NG))
    h = lax.dot_general(x, w1_ref[...], (((1,), (1,)), ((), ())),
                        preferred_element_type=jnp.float32) + b1_ref[...]
    logits_t = lax.dot_general(w2_ref[...], h, (((1,), (1,)), ((), ())),
                               preferred_element_type=jnp.float32) + b2_ref[...]
    o_ref[...] = jax.nn.sigmoid(logits_t)


_combine = pl.pallas_call(
    _combine_body,
    in_specs=[
        pl.BlockSpec((B, D), lambda: (0, 0)),
        pl.BlockSpec((B, D), lambda: (0, 0)),
        pl.BlockSpec((H, D), lambda: (0, 0)),
        pl.BlockSpec((1, H), lambda: (0, 0)),
        pl.BlockSpec((C, H), lambda: (0, 0)),
        pl.BlockSpec((C, 1), lambda: (0, 0)),
    ],
    out_specs=pl.BlockSpec((C, B), lambda: (0, 0)),
    out_shape=jax.ShapeDtypeStruct((C, B), jnp.float32),
)


def kernel(embs, ngram_embs, table, W_i2h, b_i2h, W_h2o, b_h2o):
    ng_sum = _ngram_sum(ngram_embs.astype(jnp.int32), table)
    es = _embs_sum(jnp.transpose(embs, (1, 0, 2)))
    out_t = _combine(es, ng_sum, W_i2h, b_i2h.reshape(1, H),
                     W_h2o, b_h2o.reshape(C, 1))
    return jnp.transpose(out_t)


# speedup vs baseline: 1.0495x; 1.0029x over previous
import functools

import jax
import jax.numpy as jnp
from jax import lax
from jax.experimental import pallas as pl
from jax.experimental.pallas import tpu as pltpu
from jax.experimental.pallas import tpu_sc as plsc

B = 4096
D = 128
NG = 200
WL = 50
H = 100
C = 10

NC = 2
NS = 16
NW = NC * NS
B_PER_W = B // NW
LANES = 16
NBUF = 3
CH0 = 128
CH1 = NG - CH0

_mesh = plsc.VectorSubcoreMesh(core_axis_name="c", subcore_axis_name="s")


@functools.partial(
    pl.kernel,
    out_type=jax.ShapeDtypeStruct((B, D), jnp.float32),
    mesh=_mesh,
    scratch_types=[
        pltpu.VMEM((B_PER_W, NG), jnp.int32),
        pltpu.VMEM((NBUF, NG, D), jnp.float32),
        pltpu.VMEM((B_PER_W, D), jnp.float32),
        [pltpu.SemaphoreType.DMA] * (2 * NBUF),
    ],
)
def _ngram_sum(idx_hbm, table_hbm, out_hbm, idx_v, rows_v, out_v, sems):
    wid = lax.axis_index("s") * NC + lax.axis_index("c")
    base = pl.multiple_of(wid * B_PER_W, B_PER_W)
    pltpu.sync_copy(idx_hbm.at[pl.ds(base, B_PER_W)], idx_v)

    def fire(r, h, b):
        off, n = (0, CH0) if h == 0 else (CH0, CH1)
        pltpu.async_copy(
            table_hbm.at[idx_v.at[r, pl.ds(off, n)]],
            rows_v.at[b].at[pl.ds(off, n)], sems[2 * b + h])

    def drain(h, b):
        off, n = (0, CH0) if h == 0 else (CH0, CH1)
        pltpu.make_async_copy(
            table_hbm.at[pl.ds(0, n)],
            rows_v.at[b].at[pl.ds(off, n)], sems[2 * b + h]).wait()

    def accum_chunk(b, off, n, accs):
        def body(j, accs):
            for u in range(4):
                accs = tuple(
                    accs[d] + rows_v[b, off + 4 * j + u,
                                     pl.ds(d * LANES, LANES)]
                    for d in range(D // LANES))
            return accs

        return lax.fori_loop(0, n // 4, body, accs)

    zeros = tuple(jnp.zeros((LANES,), jnp.float32) for _ in range(D // LANES))

    for b in range(NBUF):
        fire(b, 0, b)
        fire(b, 1, b)

    def step(r, b, refire):
        drain(0, b)
        accs = accum_chunk(b, 0, CH0, zeros)
        if refire:
            @pl.when(r + NBUF < B_PER_W)
            def _():
                fire(r + NBUF, 0, b)

        drain(1, b)
        accs = accum_chunk(b, CH0, CH1, accs)
        if refire:
            @pl.when(r + NBUF < B_PER_W)
            def _():
                fire(r + NBUF, 1, b)

        for d in range(D // LANES):
            out_v[r, pl.ds(d * LANES, LANES)] = accs[d]

    ROWS_PER_IT = 2 * NBUF
    N_IT = B_PER_W // ROWS_PER_IT

    def ring_body(rr, _):
        r0 = ROWS_PER_IT * rr
        for k in range(ROWS_PER_IT):
            step(r0 + k, k % NBUF, True)
        return 0

    lax.fori_loop(0, N_IT, ring_body, 0)
    for r in range(N_IT * ROWS_PER_IT, B_PER_W):
        step(r, r % NBUF, False)
    pltpu.sync_copy(out_v, out_hbm.at[pl.ds(base, B_PER_W)])


BB = 256


def _embs_sum_body(embs_ref, o_ref):
    o_ref[...] = jnp.sum(embs_ref[...], axis=0)


_embs_sum = pl.pallas_call(
    _embs_sum_body,
    grid=(B // BB,),
    in_specs=[pl.BlockSpec((WL, BB, D), lambda i: (0, i, 0))],
    out_specs=pl.BlockSpec((BB, D), lambda i: (i, 0)),
    out_shape=jax.ShapeDtypeStruct((B, D), jnp.float32),
)


def _combine_body(es_ref, ng_ref, w1_ref, b1_ref, w2_ref, b2_ref, o_ref):
    x = (es_ref[...] + ng_ref[...]) * (1.0 / (WL + NG))
    h = lax.dot_general(x, w1_ref[...], (((1,), (1,)), ((), ())),
                        preferred_element_type=jnp.float32) + b1_ref[...]
    logits_t = lax.dot_general(w2_ref[...], h, (((1,), (1,)), ((), ())),
                               preferred_element_type=jnp.float32) + b2_ref[...]
    o_ref[...] = jax.nn.sigmoid(logits_t)


_combine = pl.pallas_call(
    _combine_body,
    in_specs=[
        pl.BlockSpec((B, D), lambda: (0, 0)),
        pl.BlockSpec((B, D), lambda: (0, 0)),
        pl.BlockSpec((H, D), lambda: (0, 0)),
        pl.BlockSpec((1, H), lambda: (0, 0)),
        pl.BlockSpec((C, H), lambda: (0, 0)),
        pl.BlockSpec((C, 1), lambda: (0, 0)),
    ],
    out_specs=pl.BlockSpec((C, B), lambda: (0, 0)),
    out_shape=jax.ShapeDtypeStruct((C, B), jnp.float32),
)


def kernel(embs, ngram_embs, table, W_i2h, b_i2h, W_h2o, b_h2o):
    ng_sum = _ngram_sum(ngram_embs.astype(jnp.int32), table)
    es = _embs_sum(jnp.transpose(embs, (1, 0, 2)))
    out_t = _combine(es, ng_sum, W_i2h, b_i2h.reshape(1, H),
                     W_h2o, b_h2o.reshape(C, 1))
    return jnp.transpose(out_t)
